# Initial kernel scaffold; baseline (speedup 1.0000x reference)
import functools

import jax
import jax.numpy as jnp
from jax import lax
from jax.experimental import pallas as pl
from jax.experimental.pallas import tpu as pltpu
from jax.experimental.pallas import tpu_sc as plsc

N = 177
C = 128
E = 31329
NP = 192
EPAD = 31744
NW = 32
EPW = EPAD // NW
G = EPW // 16
F32 = jnp.float32


def _tc1_body(xs_ref, w_ref, att2_ref, xt_ref, a2_ref):
    xt = jnp.dot(xs_ref[...], w_ref[...], preferred_element_type=F32)
    xt_ref[...] = xt
    a2_ref[...] = jnp.dot(xt, att2_ref[...], preferred_element_type=F32)


def _tc1(xs, W, att2):
    return pl.pallas_call(
        _tc1_body,
        out_shape=[
            jax.ShapeDtypeStruct((NP, C), F32),
            jax.ShapeDtypeStruct((NP, C), F32),
        ],
    )(xs, W, att2)


def _take(v, i):
    return jnp.take(v, i, mode="promise_in_bounds")


def _sc_body(src_h, dst_h, ea_h, ai_h, aj_h, c_h,
             pout, qout, mout,
             src_v, dst_v, ea_v, al_v, ai_v, aj_v, c_v, m_v, p_loc, q_loc):
    wid = lax.axis_index("c") * 16 + lax.axis_index("s")
    base = wid * EPW

    pltpu.sync_copy(src_h.at[pl.ds(base, EPW)], src_v)
    pltpu.sync_copy(dst_h.at[pl.ds(base, EPW)], dst_v)
    pltpu.sync_copy(ea_h.at[pl.ds(base, EPW)], ea_v)
    pltpu.sync_copy(ai_h, ai_v)
    pltpu.sync_copy(aj_h, aj_v)
    pltpu.sync_copy(c_h, c_v)

    zeros16 = jnp.zeros((16,), F32)

    def zero_body(i, carry):
        for j in range(NP // 16):
            p_loc[i, pl.ds(j * 16, 16)] = zeros16
            q_loc[i, pl.ds(j * 16, 16)] = zeros16
        return carry

    lax.fori_loop(0, NP, zero_body, 0)

    cv = c_v[...]

    def p1_body(g, mx):
        d = dst_v[pl.ds(g * 16, 16)]
        s = src_v[pl.ds(g * 16, 16)]
        e = ea_v[pl.ds(g * 16, 16)]
        ai = plsc.load_gather(ai_v, [d])
        aj = plsc.load_gather(aj_v, [s])
        al = ai + aj + e * cv
        al = jnp.where(al >= 0.0, al, 0.2 * al)
        al_v[pl.ds(g * 16, 16)] = al
        return jnp.maximum(mx, al)

    mx = lax.fori_loop(0, G, p1_body, jnp.full((16,), -3.0e38, F32))
    m = jnp.max(mx)
    m_v[...] = jnp.broadcast_to(m, (16,))

    lane = lax.iota(jnp.int32, 16)

    def p2_body(g, carry):
        d = dst_v[pl.ds(g * 16, 16)]
        s = src_v[pl.ds(g * 16, 16)]
        e = ea_v[pl.ds(g * 16, 16)]
        al = al_v[pl.ds(g * 16, 16)]
        p = jnp.exp(al - m)
        q = p * e
        k = d * NP + s
        ks, perm = plsc.sort_key_val(k, lane)
        ps = _take(p, perm)
        qs = _take(q, perm)
        prev = _take(ks, jnp.maximum(lane - 1, 0))
        nxt = _take(ks, jnp.minimum(lane + 1, 15))
        is_start = (lane == 0) | (ks != prev)
        is_end = (lane == 15) | (ks != nxt)
        tp = plsc.cumsum(ps)
        tq = plsc.cumsum(qs)
        startlane = plsc.cummax(jnp.where(is_start, lane, 0))
        runp = tp - (_take(tp, startlane) - _take(ps, startlane))
        runq = tq - (_take(tq, startlane) - _take(qs, startlane))
        kd = ks // NP
        kc = ks - kd * NP
        plsc.addupdate_scatter(p_loc, [kd, kc], runp, mask=is_end)
        plsc.addupdate_scatter(q_loc, [kd, kc], runq, mask=is_end)
        return carry

    lax.fori_loop(0, G, p2_body, 0)

    pltpu.sync_copy(p_loc, pout.at[wid])
    pltpu.sync_copy(q_loc, qout.at[wid])
    pltpu.sync_copy(m_v, mout.at[wid])


@functools.partial(
    pl.kernel,
    out_type=[
        jax.ShapeDtypeStruct((NW, NP, NP), F32),
        jax.ShapeDtypeStruct((NW, NP, NP), F32),
        jax.ShapeDtypeStruct((NW, 16), F32),
    ],
    mesh=plsc.VectorSubcoreMesh(core_axis_name="c", subcore_axis_name="s"),
    scratch_types=[
        pltpu.VMEM((EPW,), jnp.int32),
        pltpu.VMEM((EPW,), jnp.int32),
        pltpu.VMEM((EPW,), F32),
        pltpu.VMEM((EPW,), F32),
        pltpu.VMEM((NP,), F32),
        pltpu.VMEM((NP,), F32),
        pltpu.VMEM((16,), F32),
        pltpu.VMEM((16,), F32),
        pltpu.VMEM((NP, NP), F32),
        pltpu.VMEM((NP, NP), F32),
    ],
)
def _sc_edge(src_h, dst_h, ea_h, ai_h, aj_h, c_h, pout, qout, mout,
             src_v, dst_v, ea_v, al_v, ai_v, aj_v, c_v, m_v, p_loc, q_loc):
    _sc_body(src_h, dst_h, ea_h, ai_h, aj_h, c_h, pout, qout, mout,
             src_v, dst_v, ea_v, al_v, ai_v, aj_v, c_v, m_v, p_loc, q_loc)


def _tc2_body(p_ref, q_ref, m_ref, xt_ref, u0_ref, u1_ref, b_ref, out_ref):
    mrows = jnp.max(m_ref[...], axis=1)
    g = jnp.max(mrows)
    sc = jnp.exp(mrows - g)
    sc3 = sc[:, None, None]
    P = jnp.sum(p_ref[...] * sc3, axis=0)
    Q = jnp.sum(q_ref[...] * sc3, axis=0)
    denom = jnp.sum(P, axis=1, keepdims=True) + 1e-16
    qn = jnp.sum(Q, axis=1, keepdims=True) / denom
    A = jnp.dot(P, xt_ref[...], preferred_element_type=F32) / denom
    out = jnp.dot(A, u0_ref[...], preferred_element_type=F32)
    out_ref[...] = out + qn * u1_ref[...] + b_ref[...]


def _tc2(pparts, qparts, mvec, xt, u0, u1, b):
    return pl.pallas_call(
        _tc2_body,
        out_shape=jax.ShapeDtypeStruct((NP, C), F32),
    )(pparts, qparts, mvec, xt, u0, u1, b)


def kernel(x, edge_index, edge_attr, W, att, edge_update1, bias):
    xs = jnp.zeros((NP, C), F32).at[:N].set(x[0])
    att2 = jnp.zeros((C, C), F32)
    att2 = att2.at[:, 0].set(att[0, 0, :C]).at[:, 1].set(att[0, 0, C:2 * C])

    xt, a2 = _tc1(xs, W, att2)
    a_i = a2[:, 0]
    a_j = a2[:, 1]
    c16 = jnp.full((16,), att[0, 0, 2 * C], F32)

    npad = EPAD - E
    srcp = jnp.concatenate([edge_index[0],
                            jnp.full((npad,), NP - 1, jnp.int32)])
    dstp = jnp.concatenate([edge_index[1],
                            jnp.full((npad,), NP - 1, jnp.int32)])
    eap = jnp.concatenate([edge_attr[:, 0], jnp.zeros((npad,), F32)])

    pparts, qparts, mvec = _sc_edge(srcp, dstp, eap, a_i, a_j, c16)

    out = _tc2(pparts, qparts, mvec, xt,
               edge_update1[:C], edge_update1[C:C + 1], bias[None, :])
    return out[:N].reshape(1, N, C)


# speedup vs baseline: 20.8564x; 20.8564x over previous
import functools

import jax
import jax.numpy as jnp
from jax import lax
from jax.experimental import pallas as pl
from jax.experimental.pallas import tpu as pltpu
from jax.experimental.pallas import tpu_sc as plsc

N = 177
C = 128
E = 31329
NP = 192
EPAD = 31744
NW = 32
EPW = EPAD // NW
G = EPW // 16
F32 = jnp.float32


def _tc1_body(xs_ref, w_ref, att2_ref, xt_ref, a2_ref):
    xt = jnp.dot(xs_ref[...], w_ref[...], preferred_element_type=F32)
    xt_ref[...] = xt
    a2_ref[...] = jnp.dot(xt, att2_ref[...], preferred_element_type=F32)


def _tc1(xs, W, att2):
    return pl.pallas_call(
        _tc1_body,
        out_shape=[
            jax.ShapeDtypeStruct((NP, C), F32),
            jax.ShapeDtypeStruct((NP, C), F32),
        ],
    )(xs, W, att2)


_TAKE_DNUMS = lax.GatherDimensionNumbers(
    offset_dims=(), collapsed_slice_dims=(0,), start_index_map=(0,))


def _take(v, i):
    return lax.gather(v, i[:, None], _TAKE_DNUMS, slice_sizes=(1,),
                      mode=lax.GatherScatterMode.PROMISE_IN_BOUNDS)


def _sc_body(src_h, dst_h, ea_h, ai_h, aj_h, c_h,
             pout, qout, mout,
             src_v, dst_v, ea_v, al_v, ai_v, aj_v, c_v, m_v, p_loc, q_loc):
    wid = lax.axis_index("c") * 16 + lax.axis_index("s")
    base = wid * EPW

    pltpu.sync_copy(src_h.at[pl.ds(base, EPW)], src_v)
    pltpu.sync_copy(dst_h.at[pl.ds(base, EPW)], dst_v)
    pltpu.sync_copy(ea_h.at[pl.ds(base, EPW)], ea_v)
    pltpu.sync_copy(ai_h, ai_v)
    pltpu.sync_copy(aj_h, aj_v)
    pltpu.sync_copy(c_h, c_v)

    zeros16 = jnp.zeros((16,), F32)

    def zero_body(i, carry):
        for j in range(NP // 16):
            p_loc[i, pl.ds(j * 16, 16)] = zeros16
            q_loc[i, pl.ds(j * 16, 16)] = zeros16
        return carry

    lax.fori_loop(0, NP, zero_body, 0)

    cv = c_v[...]

    def p1_body(g, mx):
        d = dst_v[pl.ds(g * 16, 16)]
        s = src_v[pl.ds(g * 16, 16)]
        e = ea_v[pl.ds(g * 16, 16)]
        ai = plsc.load_gather(ai_v, [d])
        aj = plsc.load_gather(aj_v, [s])
        al = ai + aj + e * cv
        al = jnp.where(al >= 0.0, al, 0.2 * al)
        al_v[pl.ds(g * 16, 16)] = al
        return jnp.maximum(mx, al)

    mx = lax.fori_loop(0, G, p1_body, jnp.full((16,), -3.0e38, F32))
    m = jnp.max(mx)
    m_v[...] = jnp.broadcast_to(m, (16,))

    lane = lax.iota(jnp.int32, 16)

    def p2_body(g, carry):
        d = dst_v[pl.ds(g * 16, 16)]
        s = src_v[pl.ds(g * 16, 16)]
        e = ea_v[pl.ds(g * 16, 16)]
        al = al_v[pl.ds(g * 16, 16)]
        p = jnp.exp(al - m)
        q = p * e
        k = d * NP + s
        ks, perm = plsc.sort_key_val(k, lane)
        ps = _take(p, perm)
        qs = _take(q, perm)
        prev = _take(ks, jnp.maximum(lane - 1, 0))
        nxt = _take(ks, jnp.minimum(lane + 1, 15))
        is_start = (lane == 0) | (ks != prev)
        is_end = (lane == 15) | (ks != nxt)
        tp = plsc.cumsum(ps)
        tq = plsc.cumsum(qs)
        startlane = plsc.cummax(jnp.where(is_start, lane, 0))
        runp = tp - (_take(tp, startlane) - _take(ps, startlane))
        runq = tq - (_take(tq, startlane) - _take(qs, startlane))
        kd = ks // NP
        kc = ks - kd * NP
        plsc.addupdate_scatter(p_loc, [kd, kc], runp, mask=is_end)
        plsc.addupdate_scatter(q_loc, [kd, kc], runq, mask=is_end)
        return carry

    lax.fori_loop(0, G, p2_body, 0)

    pltpu.sync_copy(p_loc, pout.at[wid])
    pltpu.sync_copy(q_loc, qout.at[wid])
    pltpu.sync_copy(m_v, mout.at[wid])


@functools.partial(
    pl.kernel,
    out_type=[
        jax.ShapeDtypeStruct((NW, NP, NP), F32),
        jax.ShapeDtypeStruct((NW, NP, NP), F32),
        jax.ShapeDtypeStruct((NW, 16), F32),
    ],
    mesh=plsc.VectorSubcoreMesh(core_axis_name="c", subcore_axis_name="s"),
    compiler_params=pltpu.CompilerParams(needs_layout_passes=False),
    scratch_types=[
        pltpu.VMEM((EPW,), jnp.int32),
        pltpu.VMEM((EPW,), jnp.int32),
        pltpu.VMEM((EPW,), F32),
        pltpu.VMEM((EPW,), F32),
        pltpu.VMEM((NP,), F32),
        pltpu.VMEM((NP,), F32),
        pltpu.VMEM((16,), F32),
        pltpu.VMEM((16,), F32),
        pltpu.VMEM((NP, NP), F32),
        pltpu.VMEM((NP, NP), F32),
    ],
)
def _sc_edge(src_h, dst_h, ea_h, ai_h, aj_h, c_h, pout, qout, mout,
             src_v, dst_v, ea_v, al_v, ai_v, aj_v, c_v, m_v, p_loc, q_loc):
    _sc_body(src_h, dst_h, ea_h, ai_h, aj_h, c_h, pout, qout, mout,
             src_v, dst_v, ea_v, al_v, ai_v, aj_v, c_v, m_v, p_loc, q_loc)


def _tc2_body(p_ref, q_ref, m_ref, xt_ref, u0_ref, u1_ref, b_ref, out_ref):
    mrows = jnp.max(m_ref[...], axis=1)
    g = jnp.max(mrows)
    sc = jnp.exp(mrows - g)
    sc3 = sc[:, None, None]
    P = jnp.sum(p_ref[...] * sc3, axis=0)
    Q = jnp.sum(q_ref[...] * sc3, axis=0)
    denom = jnp.sum(P, axis=1, keepdims=True) + 1e-16
    qn = jnp.sum(Q, axis=1, keepdims=True) / denom
    A = jnp.dot(P, xt_ref[...], preferred_element_type=F32) / denom
    out = jnp.dot(A, u0_ref[...], preferred_element_type=F32)
    out_ref[...] = out + qn * u1_ref[...] + b_ref[...]


def _tc2(pparts, qparts, mvec, xt, u0, u1, b):
    return pl.pallas_call(
        _tc2_body,
        out_shape=jax.ShapeDtypeStruct((NP, C), F32),
    )(pparts, qparts, mvec, xt, u0, u1, b)


def kernel(x, edge_index, edge_attr, W, att, edge_update1, bias):
    xs = jnp.zeros((NP, C), F32).at[:N].set(x[0])
    att2 = jnp.zeros((C, C), F32)
    att2 = att2.at[:, 0].set(att[0, 0, :C]).at[:, 1].set(att[0, 0, C:2 * C])

    xt, a2 = _tc1(xs, W, att2)
    a_i = a2[:, 0]
    a_j = a2[:, 1]
    c16 = jnp.full((16,), att[0, 0, 2 * C], F32)

    npad = EPAD - E
    srcp = jnp.concatenate([edge_index[0],
                            jnp.full((npad,), NP - 1, jnp.int32)])
    dstp = jnp.concatenate([edge_index[1],
                            jnp.full((npad,), NP - 1, jnp.int32)])
    eap = jnp.concatenate([edge_attr[:, 0], jnp.zeros((npad,), F32)])

    pparts, qparts, mvec = _sc_edge(srcp, dstp, eap, a_i, a_j, c16)

    out = _tc2(pparts, qparts, mvec, xt,
               edge_update1[:C], edge_update1[C:C + 1], bias[None, :])
    return out[:N].reshape(1, N, C)


# speedup vs baseline: 23.3154x; 1.1179x over previous
import functools

import jax
import jax.numpy as jnp
from jax import lax
from jax.experimental import pallas as pl
from jax.experimental.pallas import tpu as pltpu
from jax.experimental.pallas import tpu_sc as plsc

N = 177
C = 128
E = 31329
NP = 192
NW = 32
EPW = 992
ETAIL = E - (NW - 1) * EPW
G = EPW // 16
F32 = jnp.float32


def _tc1_body(x_ref, w_ref, attm_ref, xt_ref, a2_ref, c_ref):
    xs = jnp.concatenate([x_ref[0], jnp.zeros((NP - N, C), F32)], axis=0)
    xt = jnp.dot(xs, w_ref[...], preferred_element_type=F32)
    xt_ref[...] = xt
    attm = attm_ref[...]
    a2_ref[...] = lax.dot_general(attm[0:2, :], xt, (((1,), (1,)), ((), ())),
                                  preferred_element_type=F32)
    c_ref[...] = attm[2:3, 0:16]


def _tc1(x, W, attm):
    return pl.pallas_call(
        _tc1_body,
        out_shape=[
            jax.ShapeDtypeStruct((NP, C), F32),
            jax.ShapeDtypeStruct((2, NP), F32),
            jax.ShapeDtypeStruct((1, 16), F32),
        ],
    )(x, W, attm)


_TAKE_DNUMS = lax.GatherDimensionNumbers(
    offset_dims=(), collapsed_slice_dims=(0,), start_index_map=(0,))


def _take(v, i):
    return lax.gather(v, i[:, None], _TAKE_DNUMS, slice_sizes=(1,),
                      mode=lax.GatherScatterMode.PROMISE_IN_BOUNDS)


def _sc_body(src_h, dst_h, ea_h, a2_h, c_h,
             pout, qout, mout,
             src_v, dst_v, ea_v, al_v, ai_v, aj_v, c_v, m_v, p_loc, q_loc):
    wid = lax.axis_index("c") * 16 + lax.axis_index("s")
    base = wid * EPW

    @pl.when(wid < NW - 1)
    def _():
        pltpu.sync_copy(src_h.at[pl.ds(base, EPW)], src_v)
        pltpu.sync_copy(dst_h.at[pl.ds(base, EPW)], dst_v)
        pltpu.sync_copy(ea_h.at[pl.ds(base, EPW)], ea_v)

    @pl.when(wid == NW - 1)
    def _():
        pltpu.sync_copy(src_h.at[pl.ds(base, ETAIL)],
                        src_v.at[pl.ds(0, ETAIL)])
        pltpu.sync_copy(dst_h.at[pl.ds(base, ETAIL)],
                        dst_v.at[pl.ds(0, ETAIL)])
        pltpu.sync_copy(ea_h.at[pl.ds(base, ETAIL)],
                        ea_v.at[pl.ds(0, ETAIL)])

    pltpu.sync_copy(a2_h.at[0], ai_v)
    pltpu.sync_copy(a2_h.at[1], aj_v)
    pltpu.sync_copy(c_h.at[0], c_v)

    limit = jnp.where(wid == NW - 1, ETAIL, EPW)
    zeros16 = jnp.zeros((16,), F32)

    def zero_body(i, carry):
        for j in range(NP // 16):
            p_loc[i, pl.ds(j * 16, 16)] = zeros16
            q_loc[i, pl.ds(j * 16, 16)] = zeros16
        return carry

    lax.fori_loop(0, NP, zero_body, 0)

    cv = c_v[...]
    lane = lax.iota(jnp.int32, 16)

    def p1_body(g, mx):
        valid = (lane + g * 16) < limit
        d = jnp.where(valid, dst_v[pl.ds(g * 16, 16)], 0)
        s = jnp.where(valid, src_v[pl.ds(g * 16, 16)], 0)
        e = ea_v[pl.ds(g * 16, 16)]
        ai = plsc.load_gather(ai_v, [d])
        aj = plsc.load_gather(aj_v, [s])
        al = ai + aj + e * cv
        al = jnp.where(al >= 0.0, al, 0.2 * al)
        al_v[pl.ds(g * 16, 16)] = al
        return jnp.maximum(mx, jnp.where(valid, al, -3.0e38))

    mx = lax.fori_loop(0, G, p1_body, jnp.full((16,), -3.0e38, F32))
    m = jnp.max(mx)
    m_v[...] = jnp.broadcast_to(m, (16,))

    def p2_body(g, carry):
        valid = (lane + g * 16) < limit
        d = jnp.where(valid, dst_v[pl.ds(g * 16, 16)], 0)
        s = jnp.where(valid, src_v[pl.ds(g * 16, 16)], 0)
        e = ea_v[pl.ds(g * 16, 16)]
        al = al_v[pl.ds(g * 16, 16)]
        p0 = jnp.exp(al - m)
        p = jnp.where(valid, p0, 0.0)
        q = jnp.where(valid, p0 * e, 0.0)
        k = jnp.where(valid, d * NP + s, NP * NP - 1)
        ks, perm = plsc.sort_key_val(k, lane)
        ps = _take(p, perm)
        qs = _take(q, perm)
        prev = _take(ks, jnp.maximum(lane - 1, 0))
        nxt = _take(ks, jnp.minimum(lane + 1, 15))
        is_start = (lane == 0) | (ks != prev)
        is_end = (lane == 15) | (ks != nxt)
        tp = plsc.cumsum(ps)
        tq = plsc.cumsum(qs)
        startlane = plsc.cummax(jnp.where(is_start, lane, 0))
        runp = tp - (_take(tp, startlane) - _take(ps, startlane))
        runq = tq - (_take(tq, startlane) - _take(qs, startlane))
        kd = ks // NP
        kc = ks - kd * NP
        plsc.addupdate_scatter(p_loc, [kd, kc], runp, mask=is_end)
        plsc.addupdate_scatter(q_loc, [kd, kc], runq, mask=is_end)
        return carry

    lax.fori_loop(0, G, p2_body, 0)

    pltpu.sync_copy(p_loc, pout.at[wid])
    pltpu.sync_copy(q_loc, qout.at[wid])
    pltpu.sync_copy(m_v, mout.at[wid])


@functools.partial(
    pl.kernel,
    out_type=[
        jax.ShapeDtypeStruct((NW, NP, NP), F32),
        jax.ShapeDtypeStruct((NW, NP, NP), F32),
        jax.ShapeDtypeStruct((NW, 16), F32),
    ],
    mesh=plsc.VectorSubcoreMesh(core_axis_name="c", subcore_axis_name="s"),
    compiler_params=pltpu.CompilerParams(needs_layout_passes=False),
    scratch_types=[
        pltpu.VMEM((EPW,), jnp.int32),
        pltpu.VMEM((EPW,), jnp.int32),
        pltpu.VMEM((EPW,), F32),
        pltpu.VMEM((EPW,), F32),
        pltpu.VMEM((NP,), F32),
        pltpu.VMEM((NP,), F32),
        pltpu.VMEM((16,), F32),
        pltpu.VMEM((16,), F32),
        pltpu.VMEM((NP, NP), F32),
        pltpu.VMEM((NP, NP), F32),
    ],
)
def _sc_edge(src_h, dst_h, ea_h, a2_h, c_h, pout, qout, mout,
             src_v, dst_v, ea_v, al_v, ai_v, aj_v, c_v, m_v, p_loc, q_loc):
    _sc_body(src_h, dst_h, ea_h, a2_h, c_h, pout, qout, mout,
             src_v, dst_v, ea_v, al_v, ai_v, aj_v, c_v, m_v, p_loc, q_loc)


def _tc2_body(p_ref, q_ref, m_ref, xt_ref, eu_ref, b_ref, out_ref):
    mrows = jnp.max(m_ref[...], axis=1)
    g = jnp.max(mrows)
    sc = jnp.exp(mrows - g)
    sc3 = sc[:, None, None]
    P = jnp.sum(p_ref[...] * sc3, axis=0)
    Q = jnp.sum(q_ref[...] * sc3, axis=0)
    denom = jnp.sum(P, axis=1, keepdims=True) + 1e-16
    qn = jnp.sum(Q, axis=1, keepdims=True) / denom
    A = jnp.dot(P, xt_ref[...], preferred_element_type=F32) / denom
    eu = eu_ref[...]
    out = jnp.dot(A, eu[:C, :], preferred_element_type=F32)
    res = out + qn * eu[C:C + 1, :] + b_ref[...][None, :]
    out_ref[0] = res[:N, :]


def _tc2(pparts, qparts, mvec, xt, eu, b):
    return pl.pallas_call(
        _tc2_body,
        out_shape=jax.ShapeDtypeStruct((1, N, C), F32),
    )(pparts, qparts, mvec, xt, eu, b)


def kernel(x, edge_index, edge_attr, W, att, edge_update1, bias):
    attm = jnp.zeros((8, C), F32)
    attm = attm.at[:2].set(att[0, 0, :2 * C].reshape(2, C))
    attm = attm.at[2].set(att[0, 0, 2 * C])

    xt, a2, c2 = _tc1(x, W, attm)
    ea_flat = edge_attr.reshape(E)

    pparts, qparts, mvec = _sc_edge(edge_index[0], edge_index[1], ea_flat, a2, c2)

    return _tc2(pparts, qparts, mvec, xt, edge_update1, bias)


# speedup vs baseline: 25.0151x; 1.0729x over previous
import functools

import jax
import jax.numpy as jnp
from jax import lax
from jax.experimental import pallas as pl
from jax.experimental.pallas import tpu as pltpu
from jax.experimental.pallas import tpu_sc as plsc

N = 177
C = 128
E = 31329
NP = 192
NW = 32
EPW = 992
ETAIL = E - (NW - 1) * EPW
G = EPW // 16
F32 = jnp.float32


def _tc1_body(x_ref, w_ref, attm_ref, xt_ref, a2_ref, c_ref):
    xs = jnp.concatenate([x_ref[0], jnp.zeros((NP - N, C), F32)], axis=0)
    xt = jnp.dot(xs, w_ref[...], preferred_element_type=F32)
    xt_ref[...] = xt
    attm = attm_ref[...]
    a2_ref[...] = lax.dot_general(attm[0:2, :], xt, (((1,), (1,)), ((), ())),
                                  preferred_element_type=F32)
    c_ref[...] = attm[2:3, 0:16]


def _tc1(x, W, attm):
    return pl.pallas_call(
        _tc1_body,
        out_shape=[
            jax.ShapeDtypeStruct((NP, C), F32),
            jax.ShapeDtypeStruct((2, NP), F32),
            jax.ShapeDtypeStruct((1, 16), F32),
        ],
    )(x, W, attm)


_TAKE_DNUMS = lax.GatherDimensionNumbers(
    offset_dims=(), collapsed_slice_dims=(0,), start_index_map=(0,))


def _take(v, i):
    return lax.gather(v, i[:, None], _TAKE_DNUMS, slice_sizes=(1,),
                      mode=lax.GatherScatterMode.PROMISE_IN_BOUNDS)


def _sc_body(src_h, dst_h, ea_h, a2_h, c_h,
             pout, qout, mout,
             src_v, dst_v, ea_v, al_v, ai_v, aj_v, c_v, m_v, p_loc, q_loc,
             sem):
    wid = lax.axis_index("c") * 16 + lax.axis_index("s")
    base = wid * EPW

    @pl.when(wid < NW - 1)
    def _():
        pltpu.async_copy(src_h.at[pl.ds(base, EPW)], src_v, sem)
        pltpu.async_copy(dst_h.at[pl.ds(base, EPW)], dst_v, sem)
        pltpu.async_copy(ea_h.at[pl.ds(base, EPW)], ea_v, sem)

    @pl.when(wid == NW - 1)
    def _():
        pltpu.async_copy(src_h.at[pl.ds(base, ETAIL)],
                         src_v.at[pl.ds(0, ETAIL)], sem)
        pltpu.async_copy(dst_h.at[pl.ds(base, ETAIL)],
                         dst_v.at[pl.ds(0, ETAIL)], sem)
        pltpu.async_copy(ea_h.at[pl.ds(base, ETAIL)],
                         ea_v.at[pl.ds(0, ETAIL)], sem)

    pltpu.async_copy(a2_h.at[0], ai_v, sem)
    pltpu.async_copy(a2_h.at[1], aj_v, sem)
    pltpu.async_copy(c_h.at[0], c_v, sem)

    limit = jnp.where(wid == NW - 1, ETAIL, EPW)
    zeros16 = jnp.zeros((16,), F32)

    def zero_body(i, carry):
        for j in range(NP // 16):
            p_loc[i, pl.ds(j * 16, 16)] = zeros16
            q_loc[i, pl.ds(j * 16, 16)] = zeros16
        return carry

    lax.fori_loop(0, NP, zero_body, 0, unroll=2)

    @pl.when(wid < NW - 1)
    def _():
        pltpu.make_async_copy(src_h.at[pl.ds(base, EPW)], src_v, sem).wait()
        pltpu.make_async_copy(dst_h.at[pl.ds(base, EPW)], dst_v, sem).wait()
        pltpu.make_async_copy(ea_h.at[pl.ds(base, EPW)], ea_v, sem).wait()

    @pl.when(wid == NW - 1)
    def _():
        pltpu.make_async_copy(src_h.at[pl.ds(base, ETAIL)],
                              src_v.at[pl.ds(0, ETAIL)], sem).wait()
        pltpu.make_async_copy(dst_h.at[pl.ds(base, ETAIL)],
                              dst_v.at[pl.ds(0, ETAIL)], sem).wait()
        pltpu.make_async_copy(ea_h.at[pl.ds(base, ETAIL)],
                              ea_v.at[pl.ds(0, ETAIL)], sem).wait()

    pltpu.make_async_copy(a2_h.at[0], ai_v, sem).wait()
    pltpu.make_async_copy(a2_h.at[1], aj_v, sem).wait()
    pltpu.make_async_copy(c_h.at[0], c_v, sem).wait()

    cv = c_v[...]
    lane = lax.iota(jnp.int32, 16)

    def p1_body(g, mx):
        valid = (lane + g * 16) < limit
        d = jnp.where(valid, dst_v[pl.ds(g * 16, 16)], 0)
        s = jnp.where(valid, src_v[pl.ds(g * 16, 16)], 0)
        e = ea_v[pl.ds(g * 16, 16)]
        ai = plsc.load_gather(ai_v, [d])
        aj = plsc.load_gather(aj_v, [s])
        al = ai + aj + e * cv
        al = jnp.where(al >= 0.0, al, 0.2 * al)
        al_v[pl.ds(g * 16, 16)] = al
        return jnp.maximum(mx, jnp.where(valid, al, -3.0e38))

    mx = lax.fori_loop(0, G, p1_body, jnp.full((16,), -3.0e38, F32),
                       unroll=2)
    m = jnp.max(mx)
    m_v[...] = jnp.broadcast_to(m, (16,))

    def p2_body(g, carry):
        valid = (lane + g * 16) < limit
        d = jnp.where(valid, dst_v[pl.ds(g * 16, 16)], 0)
        s = jnp.where(valid, src_v[pl.ds(g * 16, 16)], 0)
        e = ea_v[pl.ds(g * 16, 16)]
        al = al_v[pl.ds(g * 16, 16)]
        p0 = jnp.exp(al - m)
        p = jnp.where(valid, p0, 0.0)
        q = jnp.where(valid, p0 * e, 0.0)
        k = jnp.where(valid, d * NP + s, NP * NP - 1)
        ks, perm = plsc.sort_key_val(k, lane)
        ps = _take(p, perm)
        qs = _take(q, perm)
        prev = _take(ks, jnp.maximum(lane - 1, 0))
        nxt = _take(ks, jnp.minimum(lane + 1, 15))
        is_start = (lane == 0) | (ks != prev)
        is_end = (lane == 15) | (ks != nxt)
        tp = plsc.cumsum(ps)
        tq = plsc.cumsum(qs)
        startlane = plsc.cummax(jnp.where(is_start, lane, 0))
        runp = tp - (_take(tp, startlane) - _take(ps, startlane))
        runq = tq - (_take(tq, startlane) - _take(qs, startlane))
        kd = ks // NP
        kc = ks - kd * NP
        plsc.addupdate_scatter(p_loc, [kd, kc], runp, mask=is_end)
        plsc.addupdate_scatter(q_loc, [kd, kc], runq, mask=is_end)
        return carry

    lax.fori_loop(0, G, p2_body, 0, unroll=2)

    pltpu.sync_copy(p_loc, pout.at[wid])
    pltpu.sync_copy(q_loc, qout.at[wid])
    pltpu.sync_copy(m_v, mout.at[wid])


@functools.partial(
    pl.kernel,
    out_type=[
        jax.ShapeDtypeStruct((NW, NP, NP), F32),
        jax.ShapeDtypeStruct((NW, NP, NP), F32),
        jax.ShapeDtypeStruct((NW, 16), F32),
    ],
    mesh=plsc.VectorSubcoreMesh(core_axis_name="c", subcore_axis_name="s"),
    compiler_params=pltpu.CompilerParams(needs_layout_passes=False),
    scratch_types=[
        pltpu.VMEM((EPW,), jnp.int32),
        pltpu.VMEM((EPW,), jnp.int32),
        pltpu.VMEM((EPW,), F32),
        pltpu.VMEM((EPW,), F32),
        pltpu.VMEM((NP,), F32),
        pltpu.VMEM((NP,), F32),
        pltpu.VMEM((16,), F32),
        pltpu.VMEM((16,), F32),
        pltpu.VMEM((NP, NP), F32),
        pltpu.VMEM((NP, NP), F32),
        pltpu.SemaphoreType.DMA,
    ],
)
def _sc_edge(src_h, dst_h, ea_h, a2_h, c_h, pout, qout, mout,
             src_v, dst_v, ea_v, al_v, ai_v, aj_v, c_v, m_v, p_loc, q_loc,
             sem):
    _sc_body(src_h, dst_h, ea_h, a2_h, c_h, pout, qout, mout,
             src_v, dst_v, ea_v, al_v, ai_v, aj_v, c_v, m_v, p_loc, q_loc,
             sem)


def _tc2_body(p_ref, q_ref, m_ref, xt_ref, eu_ref, b_ref, out_ref):
    mrows = jnp.max(m_ref[...], axis=1)
    g = jnp.max(mrows)
    sc = jnp.exp(mrows - g)
    sc3 = sc[:, None, None]
    P = jnp.sum(p_ref[...] * sc3, axis=0)
    Q = jnp.sum(q_ref[...] * sc3, axis=0)
    denom = jnp.sum(P, axis=1, keepdims=True) + 1e-16
    qn = jnp.sum(Q, axis=1, keepdims=True) / denom
    A = jnp.dot(P, xt_ref[...], preferred_element_type=F32) / denom
    eu = eu_ref[...]
    out = jnp.dot(A, eu[:C, :], preferred_element_type=F32)
    res = out + qn * eu[C:C + 1, :] + b_ref[...][None, :]
    out_ref[0] = res[:N, :]


def _tc2(pparts, qparts, mvec, xt, eu, b):
    return pl.pallas_call(
        _tc2_body,
        out_shape=jax.ShapeDtypeStruct((1, N, C), F32),
    )(pparts, qparts, mvec, xt, eu, b)


def kernel(x, edge_index, edge_attr, W, att, edge_update1, bias):
    attm = jnp.zeros((8, C), F32)
    attm = attm.at[:2].set(att[0, 0, :2 * C].reshape(2, C))
    attm = attm.at[2].set(att[0, 0, 2 * C])

    xt, a2, c2 = _tc1(x, W, attm)
    ea_flat = edge_attr.reshape(E)

    pparts, qparts, mvec = _sc_edge(edge_index[0], edge_index[1], ea_flat, a2, c2)

    return _tc2(pparts, qparts, mvec, xt, edge_update1, bias)


# speedup vs baseline: 27.4551x; 1.0975x over previous
import functools

import jax
import jax.numpy as jnp
from jax import lax
from jax.experimental import pallas as pl
from jax.experimental.pallas import tpu as pltpu
from jax.experimental.pallas import tpu_sc as plsc

N = 177
C = 128
E = 31329
NP = 192
NW = 32
EPW = 1024
NFULL = 30
ETAIL = E - NFULL * EPW
ETDMA = 640
G = EPW // 16
F32 = jnp.float32


def _tc1_body(x_ref, w_ref, attm_ref, xt_ref, a2_ref, c_ref):
    xs = jnp.concatenate([x_ref[0], jnp.zeros((NP - N, C), F32)], axis=0)
    xt = jnp.dot(xs, w_ref[...], preferred_element_type=F32)
    xt_ref[...] = xt
    attm = attm_ref[...]
    a2_ref[...] = lax.dot_general(attm[0:2, :], xt, (((1,), (1,)), ((), ())),
                                  preferred_element_type=F32)
    c_ref[...] = attm[2:3, 0:16]


def _tc1(x, W, attm):
    return pl.pallas_call(
        _tc1_body,
        out_shape=[
            jax.ShapeDtypeStruct((NP, C), F32),
            jax.ShapeDtypeStruct((2, NP), F32),
            jax.ShapeDtypeStruct((1, 16), F32),
        ],
    )(x, W, attm)


_TAKE_DNUMS = lax.GatherDimensionNumbers(
    offset_dims=(), collapsed_slice_dims=(0,), start_index_map=(0,))


def _take(v, i):
    return lax.gather(v, i[:, None], _TAKE_DNUMS, slice_sizes=(1,),
                      mode=lax.GatherScatterMode.PROMISE_IN_BOUNDS)


def _sc_body(ei_h, ea_h, a2_h, c_h,
             pout, qout, mout,
             src_v, dst_v, ea_v, al_v, ai_v, aj_v, c_v, m_v, p_loc, q_loc,
             sem):
    wid = lax.axis_index("c") * 16 + lax.axis_index("s")
    base = wid * EPW

    srcrow = ei_h.at[0]
    dstrow = ei_h.at[1]
    earow = ea_h.at[0]

    @pl.when(wid < NFULL)
    def _():
        pltpu.async_copy(srcrow.at[pl.ds(base, EPW)], src_v, sem)
        pltpu.async_copy(dstrow.at[pl.ds(base, EPW)], dst_v, sem)
        pltpu.async_copy(earow.at[pl.ds(base, EPW)], ea_v, sem)

    @pl.when(wid == NFULL)
    def _():
        pltpu.async_copy(srcrow.at[pl.ds(base, ETDMA)],
                         src_v.at[pl.ds(0, ETDMA)], sem)
        pltpu.async_copy(dstrow.at[pl.ds(base, ETDMA)],
                         dst_v.at[pl.ds(0, ETDMA)], sem)
        pltpu.async_copy(earow.at[pl.ds(base, ETDMA)],
                         ea_v.at[pl.ds(0, ETDMA)], sem)

    pltpu.async_copy(a2_h.at[0], ai_v, sem)
    pltpu.async_copy(a2_h.at[1], aj_v, sem)
    pltpu.async_copy(c_h.at[0], c_v, sem)

    limit = jnp.where(wid < NFULL, EPW, jnp.where(wid == NFULL, ETAIL, 0))
    zeros16 = jnp.zeros((16,), F32)

    def zero_body(i, carry):
        for j in range(NP // 16):
            p_loc[i, pl.ds(j * 16, 16)] = zeros16
            q_loc[i, pl.ds(j * 16, 16)] = zeros16
        return carry

    lax.fori_loop(0, NP, zero_body, 0, unroll=2)

    @pl.when(wid < NFULL)
    def _():
        pltpu.make_async_copy(srcrow.at[pl.ds(base, EPW)], src_v, sem).wait()
        pltpu.make_async_copy(dstrow.at[pl.ds(base, EPW)], dst_v, sem).wait()
        pltpu.make_async_copy(earow.at[pl.ds(base, EPW)], ea_v, sem).wait()

    @pl.when(wid == NFULL)
    def _():
        pltpu.make_async_copy(srcrow.at[pl.ds(base, ETDMA)],
                              src_v.at[pl.ds(0, ETDMA)], sem).wait()
        pltpu.make_async_copy(dstrow.at[pl.ds(base, ETDMA)],
                              dst_v.at[pl.ds(0, ETDMA)], sem).wait()
        pltpu.make_async_copy(earow.at[pl.ds(base, ETDMA)],
                              ea_v.at[pl.ds(0, ETDMA)], sem).wait()

    pltpu.make_async_copy(a2_h.at[0], ai_v, sem).wait()
    pltpu.make_async_copy(a2_h.at[1], aj_v, sem).wait()
    pltpu.make_async_copy(c_h.at[0], c_v, sem).wait()

    cv = c_v[...]
    lane = lax.iota(jnp.int32, 16)

    def p1_body(g, mx):
        valid = (lane + g * 16) < limit
        d = jnp.where(valid, dst_v[pl.ds(g * 16, 16)], 0)
        s = jnp.where(valid, src_v[pl.ds(g * 16, 16)], 0)
        e = ea_v[pl.ds(g * 16, 16)]
        ai = plsc.load_gather(ai_v, [d])
        aj = plsc.load_gather(aj_v, [s])
        al = ai + aj + e * cv
        al = jnp.where(al >= 0.0, al, 0.2 * al)
        al_v[pl.ds(g * 16, 16)] = al
        return jnp.maximum(mx, jnp.where(valid, al, -3.0e38))

    mx = lax.fori_loop(0, G, p1_body, jnp.full((16,), -3.0e38, F32),
                       unroll=2)
    m = jnp.max(mx)
    m_v[...] = jnp.broadcast_to(m, (16,))

    def p2_body(g, carry):
        valid = (lane + g * 16) < limit
        d = jnp.where(valid, dst_v[pl.ds(g * 16, 16)], 0)
        s = jnp.where(valid, src_v[pl.ds(g * 16, 16)], 0)
        e = ea_v[pl.ds(g * 16, 16)]
        al = al_v[pl.ds(g * 16, 16)]
        p0 = jnp.exp(al - m)
        p = jnp.where(valid, p0, 0.0)
        q = jnp.where(valid, p0 * e, 0.0)
        k = jnp.where(valid, d * NP + s, NP * NP - 1)
        ks, perm = plsc.sort_key_val(k, lane)
        ps = _take(p, perm)
        qs = _take(q, perm)
        prev = _take(ks, jnp.maximum(lane - 1, 0))
        nxt = _take(ks, jnp.minimum(lane + 1, 15))
        is_start = (lane == 0) | (ks != prev)
        is_end = (lane == 15) | (ks != nxt)
        tp = plsc.cumsum(ps)
        tq = plsc.cumsum(qs)
        startlane = plsc.cummax(jnp.where(is_start, lane, 0))
        runp = tp - (_take(tp, startlane) - _take(ps, startlane))
        runq = tq - (_take(tq, startlane) - _take(qs, startlane))
        kd = ks // NP
        kc = ks - kd * NP
        plsc.addupdate_scatter(p_loc, [kd, kc], runp, mask=is_end)
        plsc.addupdate_scatter(q_loc, [kd, kc], runq, mask=is_end)
        return carry

    lax.fori_loop(0, G, p2_body, 0, unroll=2)

    pltpu.sync_copy(p_loc, pout.at[wid])
    pltpu.sync_copy(q_loc, qout.at[wid])
    pltpu.sync_copy(m_v, mout.at[wid])


@functools.partial(
    pl.kernel,
    out_type=[
        jax.ShapeDtypeStruct((NW, NP, NP), F32),
        jax.ShapeDtypeStruct((NW, NP, NP), F32),
        jax.ShapeDtypeStruct((NW, 16), F32),
    ],
    mesh=plsc.VectorSubcoreMesh(core_axis_name="c", subcore_axis_name="s"),
    compiler_params=pltpu.CompilerParams(needs_layout_passes=False),
    scratch_types=[
        pltpu.VMEM((EPW,), jnp.int32),
        pltpu.VMEM((EPW,), jnp.int32),
        pltpu.VMEM((EPW,), F32),
        pltpu.VMEM((EPW,), F32),
        pltpu.VMEM((NP,), F32),
        pltpu.VMEM((NP,), F32),
        pltpu.VMEM((16,), F32),
        pltpu.VMEM((16,), F32),
        pltpu.VMEM((NP, NP), F32),
        pltpu.VMEM((NP, NP), F32),
        pltpu.SemaphoreType.DMA,
    ],
)
def _sc_edge(ei_h, ea_h, a2_h, c_h, pout, qout, mout,
             src_v, dst_v, ea_v, al_v, ai_v, aj_v, c_v, m_v, p_loc, q_loc,
             sem):
    _sc_body(ei_h, ea_h, a2_h, c_h, pout, qout, mout,
             src_v, dst_v, ea_v, al_v, ai_v, aj_v, c_v, m_v, p_loc, q_loc,
             sem)


def _tc2_body(p_ref, q_ref, m_ref, xt_ref, eu_ref, b_ref, out_ref):
    mrows = jnp.max(m_ref[...], axis=1)
    g = jnp.max(mrows)
    sc = jnp.exp(mrows - g)
    sc3 = sc[:, None, None]
    P = jnp.sum(p_ref[...] * sc3, axis=0)
    Q = jnp.sum(q_ref[...] * sc3, axis=0)
    denom = jnp.sum(P, axis=1, keepdims=True) + 1e-16
    qn = jnp.sum(Q, axis=1, keepdims=True) / denom
    A = jnp.dot(P, xt_ref[...], preferred_element_type=F32) / denom
    eu = eu_ref[...]
    out = jnp.dot(A, eu[:C, :], preferred_element_type=F32)
    res = out + qn * eu[C:C + 1, :] + b_ref[...][None, :]
    out_ref[0] = res[:N, :]


def _tc2(pparts, qparts, mvec, xt, eu, b):
    return pl.pallas_call(
        _tc2_body,
        out_shape=jax.ShapeDtypeStruct((1, N, C), F32),
    )(pparts, qparts, mvec, xt, eu, b)


def kernel(x, edge_index, edge_attr, W, att, edge_update1, bias):
    attm = jnp.zeros((8, C), F32)
    attm = attm.at[:2].set(att[0, 0, :2 * C].reshape(2, C))
    attm = attm.at[2].set(att[0, 0, 2 * C])

    xt, a2, c2 = _tc1(x, W, attm)
    ea_row = edge_attr.mT

    pparts, qparts, mvec = _sc_edge(edge_index, ea_row, a2, c2)

    return _tc2(pparts, qparts, mvec, xt, edge_update1, bias)


# speedup vs baseline: 29.2536x; 1.0655x over previous
import functools

import jax
import jax.numpy as jnp
from jax import lax
from jax.experimental import pallas as pl
from jax.experimental.pallas import tpu as pltpu
from jax.experimental.pallas import tpu_sc as plsc

N = 177
C = 128
E = 31329
NP = 192
NW = 32
EPW = 1024
NFULL = 30
ETAIL = E - NFULL * EPW
ETDMA = 640
G = EPW // 16
F32 = jnp.float32


def _tc1_body(x_ref, w_ref, att_ref, xt_ref, a2_ref, c_ref):
    xs = jnp.concatenate([x_ref[0], jnp.zeros((NP - N, C), F32)], axis=0)
    xt = jnp.dot(xs, w_ref[...], preferred_element_type=F32)
    xt_ref[...] = xt
    attm = jnp.concatenate([att_ref[0, :, 0:C], att_ref[0, :, C:2 * C]],
                           axis=0)
    a2_ref[...] = lax.dot_general(attm, xt, (((1,), (1,)), ((), ())),
                                  preferred_element_type=F32)
    c_ref[...] = jnp.broadcast_to(att_ref[0, :, 2 * C:2 * C + 1], (1, 16))


def _tc1(x, W, att):
    return pl.pallas_call(
        _tc1_body,
        out_shape=[
            jax.ShapeDtypeStruct((NP, C), F32),
            jax.ShapeDtypeStruct((2, NP), F32),
            jax.ShapeDtypeStruct((1, 16), F32),
        ],
    )(x, W, att)


_TAKE_DNUMS = lax.GatherDimensionNumbers(
    offset_dims=(), collapsed_slice_dims=(0,), start_index_map=(0,))


def _take(v, i):
    return lax.gather(v, i[:, None], _TAKE_DNUMS, slice_sizes=(1,),
                      mode=lax.GatherScatterMode.PROMISE_IN_BOUNDS)


def _sc_body(ei_h, ea_h, a2_h, c_h,
             pout, qout, mout,
             src_v, dst_v, ea_v, al_v, ai_v, aj_v, c_v, m_v, p_loc, q_loc,
             sem):
    wid = lax.axis_index("c") * 16 + lax.axis_index("s")
    base = wid * EPW

    srcrow = ei_h.at[0]
    dstrow = ei_h.at[1]
    earow = ea_h.at[0]

    @pl.when(wid < NFULL)
    def _():
        pltpu.async_copy(srcrow.at[pl.ds(base, EPW)], src_v, sem)
        pltpu.async_copy(dstrow.at[pl.ds(base, EPW)], dst_v, sem)
        pltpu.async_copy(earow.at[pl.ds(base, EPW)], ea_v, sem)

    @pl.when(wid == NFULL)
    def _():
        pltpu.async_copy(srcrow.at[pl.ds(base, ETDMA)],
                         src_v.at[pl.ds(0, ETDMA)], sem)
        pltpu.async_copy(dstrow.at[pl.ds(base, ETDMA)],
                         dst_v.at[pl.ds(0, ETDMA)], sem)
        pltpu.async_copy(earow.at[pl.ds(base, ETDMA)],
                         ea_v.at[pl.ds(0, ETDMA)], sem)

    pltpu.async_copy(a2_h.at[0], ai_v, sem)
    pltpu.async_copy(a2_h.at[1], aj_v, sem)
    pltpu.async_copy(c_h.at[0], c_v, sem)

    limit = jnp.where(wid < NFULL, EPW, jnp.where(wid == NFULL, ETAIL, 0))
    zeros16 = jnp.zeros((16,), F32)

    def zero_body(i, carry):
        for j in range(NP // 16):
            p_loc[i, pl.ds(j * 16, 16)] = zeros16
            q_loc[i, pl.ds(j * 16, 16)] = zeros16
        return carry

    lax.fori_loop(0, NP, zero_body, 0, unroll=4)

    @pl.when(wid < NFULL)
    def _():
        pltpu.make_async_copy(srcrow.at[pl.ds(base, EPW)], src_v, sem).wait()
        pltpu.make_async_copy(dstrow.at[pl.ds(base, EPW)], dst_v, sem).wait()
        pltpu.make_async_copy(earow.at[pl.ds(base, EPW)], ea_v, sem).wait()

    @pl.when(wid == NFULL)
    def _():
        pltpu.make_async_copy(srcrow.at[pl.ds(base, ETDMA)],
                              src_v.at[pl.ds(0, ETDMA)], sem).wait()
        pltpu.make_async_copy(dstrow.at[pl.ds(base, ETDMA)],
                              dst_v.at[pl.ds(0, ETDMA)], sem).wait()
        pltpu.make_async_copy(earow.at[pl.ds(base, ETDMA)],
                              ea_v.at[pl.ds(0, ETDMA)], sem).wait()

    pltpu.make_async_copy(a2_h.at[0], ai_v, sem).wait()
    pltpu.make_async_copy(a2_h.at[1], aj_v, sem).wait()
    pltpu.make_async_copy(c_h.at[0], c_v, sem).wait()

    cv = c_v[...]
    lane = lax.iota(jnp.int32, 16)

    def p1_body(g, mx):
        valid = (lane + g * 16) < limit
        d = jnp.where(valid, dst_v[pl.ds(g * 16, 16)], 0)
        s = jnp.where(valid, src_v[pl.ds(g * 16, 16)], 0)
        e = ea_v[pl.ds(g * 16, 16)]
        ai = plsc.load_gather(ai_v, [d])
        aj = plsc.load_gather(aj_v, [s])
        al = ai + aj + e * cv
        al = jnp.where(al >= 0.0, al, 0.2 * al)
        al_v[pl.ds(g * 16, 16)] = al
        return jnp.maximum(mx, jnp.where(valid, al, -3.0e38))

    mx = lax.fori_loop(0, G, p1_body, jnp.full((16,), -3.0e38, F32),
                       unroll=4)
    m = jnp.max(mx)
    m_v[...] = jnp.broadcast_to(m, (16,))

    def p2_body(g, carry):
        valid = (lane + g * 16) < limit
        d = jnp.where(valid, dst_v[pl.ds(g * 16, 16)], 0)
        s = jnp.where(valid, src_v[pl.ds(g * 16, 16)], 0)
        e = ea_v[pl.ds(g * 16, 16)]
        al = al_v[pl.ds(g * 16, 16)]
        p0 = jnp.exp(al - m)
        p = jnp.where(valid, p0, 0.0)
        q = jnp.where(valid, p0 * e, 0.0)
        k = jnp.where(valid, d * NP + s, NP * NP - 1)
        ks, perm = plsc.sort_key_val(k, lane)
        ps = _take(p, perm)
        qs = _take(q, perm)
        prev = _take(ks, jnp.maximum(lane - 1, 0))
        nxt = _take(ks, jnp.minimum(lane + 1, 15))
        is_start = (lane == 0) | (ks != prev)
        is_end = (lane == 15) | (ks != nxt)
        tp = plsc.cumsum(ps)
        tq = plsc.cumsum(qs)
        startlane = plsc.cummax(jnp.where(is_start, lane, 0))
        runp = tp - (_take(tp, startlane) - _take(ps, startlane))
        runq = tq - (_take(tq, startlane) - _take(qs, startlane))
        kd = ks // NP
        kc = ks - kd * NP
        plsc.addupdate_scatter(p_loc, [kd, kc], runp, mask=is_end)
        plsc.addupdate_scatter(q_loc, [kd, kc], runq, mask=is_end)
        return carry

    lax.fori_loop(0, G, p2_body, 0, unroll=4)

    pltpu.sync_copy(p_loc, pout.at[wid])
    pltpu.sync_copy(q_loc, qout.at[wid])
    pltpu.sync_copy(m_v, mout.at[wid])


@functools.partial(
    pl.kernel,
    out_type=[
        jax.ShapeDtypeStruct((NW, NP, NP), F32),
        jax.ShapeDtypeStruct((NW, NP, NP), F32),
        jax.ShapeDtypeStruct((NW, 16), F32),
    ],
    mesh=plsc.VectorSubcoreMesh(core_axis_name="c", subcore_axis_name="s"),
    compiler_params=pltpu.CompilerParams(needs_layout_passes=False),
    scratch_types=[
        pltpu.VMEM((EPW,), jnp.int32),
        pltpu.VMEM((EPW,), jnp.int32),
        pltpu.VMEM((EPW,), F32),
        pltpu.VMEM((EPW,), F32),
        pltpu.VMEM((NP,), F32),
        pltpu.VMEM((NP,), F32),
        pltpu.VMEM((16,), F32),
        pltpu.VMEM((16,), F32),
        pltpu.VMEM((NP, NP), F32),
        pltpu.VMEM((NP, NP), F32),
        pltpu.SemaphoreType.DMA,
    ],
)
def _sc_edge(ei_h, ea_h, a2_h, c_h, pout, qout, mout,
             src_v, dst_v, ea_v, al_v, ai_v, aj_v, c_v, m_v, p_loc, q_loc,
             sem):
    _sc_body(ei_h, ea_h, a2_h, c_h, pout, qout, mout,
             src_v, dst_v, ea_v, al_v, ai_v, aj_v, c_v, m_v, p_loc, q_loc,
             sem)


def _tc2_body(p_ref, q_ref, m_ref, xt_ref, eu_ref, b_ref, out_ref):
    mrows = jnp.max(m_ref[...], axis=1)
    g = jnp.max(mrows)
    sc = jnp.exp(mrows - g)
    sc3 = sc[:, None, None]
    P = jnp.sum(p_ref[...] * sc3, axis=0)
    Q = jnp.sum(q_ref[...] * sc3, axis=0)
    denom = jnp.sum(P, axis=1, keepdims=True) + 1e-16
    qn = jnp.sum(Q, axis=1, keepdims=True) / denom
    A = jnp.dot(P, xt_ref[...], preferred_element_type=F32) / denom
    eu = eu_ref[...]
    out = jnp.dot(A, eu[:C, :], preferred_element_type=F32)
    res = out + qn * eu[C:C + 1, :] + b_ref[...][None, :]
    out_ref[0] = res[:N, :]


def _tc2(pparts, qparts, mvec, xt, eu, b):
    return pl.pallas_call(
        _tc2_body,
        out_shape=jax.ShapeDtypeStruct((1, N, C), F32),
    )(pparts, qparts, mvec, xt, eu, b)


def kernel(x, edge_index, edge_attr, W, att, edge_update1, bias):
    xt, a2, c2 = _tc1(x, W, att)
    ea_row = edge_attr.mT

    pparts, qparts, mvec = _sc_edge(edge_index, ea_row, a2, c2)

    return _tc2(pparts, qparts, mvec, xt, edge_update1, bias)


# speedup vs baseline: 30.4009x; 1.0392x over previous
import functools

import jax
import jax.numpy as jnp
from jax import lax
from jax.experimental import pallas as pl
from jax.experimental.pallas import tpu as pltpu
from jax.experimental.pallas import tpu_sc as plsc

N = 177
C = 128
E = 31329
NP = 192
NW = 32
EPW = 1024
NFULL = 30
ETAIL = E - NFULL * EPW
ETDMA = 640
G = EPW // 16
F32 = jnp.float32


def _tc1_body(x_ref, w_ref, att_ref, ea_ref, xt_ref, a2_ref, c_ref):
    xs = jnp.concatenate([x_ref[0], jnp.zeros((NP - N, C), F32)], axis=0)
    xt = jnp.dot(xs, w_ref[...], preferred_element_type=F32)
    xt_ref[...] = xt
    attm = jnp.concatenate([att_ref[0, :, 0:C], att_ref[0, :, C:2 * C]],
                           axis=0)
    a2 = lax.dot_general(attm, xt, (((1,), (1,)), ((), ())),
                         preferred_element_type=F32)
    a2_ref[...] = a2
    cval = att_ref[0, :, 2 * C:2 * C + 1]
    bnd = (jnp.max(a2[0:1, :]) + jnp.max(a2[1:2, :])
           + jnp.max(cval * ea_ref[...]))
    bnd = jnp.where(bnd >= 0.0, bnd, 0.2 * bnd)
    c_ref[...] = jnp.concatenate(
        [jnp.broadcast_to(cval, (1, 16)),
         jnp.broadcast_to(bnd, (1, 16))], axis=0)


def _tc1(x, W, att, ea_row):
    return pl.pallas_call(
        _tc1_body,
        out_shape=[
            jax.ShapeDtypeStruct((NP, C), F32),
            jax.ShapeDtypeStruct((2, NP), F32),
            jax.ShapeDtypeStruct((2, 16), F32),
        ],
    )(x, W, att, ea_row)


_TAKE_DNUMS = lax.GatherDimensionNumbers(
    offset_dims=(), collapsed_slice_dims=(0,), start_index_map=(0,))


def _take(v, i):
    return lax.gather(v, i[:, None], _TAKE_DNUMS, slice_sizes=(1,),
                      mode=lax.GatherScatterMode.PROMISE_IN_BOUNDS)


def _sc_body(ei_h, ea_h, a2_h, c_h,
             pout, qout,
             src_v, dst_v, ea_v, ai_v, aj_v, c_v, b_v, p_loc, q_loc,
             sem):
    wid = lax.axis_index("c") * 16 + lax.axis_index("s")
    base = wid * EPW

    srcrow = ei_h.at[0]
    dstrow = ei_h.at[1]
    earow = ea_h.at[0]

    @pl.when(wid < NFULL)
    def _():
        pltpu.async_copy(srcrow.at[pl.ds(base, EPW)], src_v, sem)
        pltpu.async_copy(dstrow.at[pl.ds(base, EPW)], dst_v, sem)
        pltpu.async_copy(earow.at[pl.ds(base, EPW)], ea_v, sem)

    @pl.when(wid == NFULL)
    def _():
        pltpu.async_copy(srcrow.at[pl.ds(base, ETDMA)],
                         src_v.at[pl.ds(0, ETDMA)], sem)
        pltpu.async_copy(dstrow.at[pl.ds(base, ETDMA)],
                         dst_v.at[pl.ds(0, ETDMA)], sem)
        pltpu.async_copy(earow.at[pl.ds(base, ETDMA)],
                         ea_v.at[pl.ds(0, ETDMA)], sem)

    pltpu.async_copy(a2_h.at[0], ai_v, sem)
    pltpu.async_copy(a2_h.at[1], aj_v, sem)
    pltpu.async_copy(c_h.at[0], c_v, sem)
    pltpu.async_copy(c_h.at[1], b_v, sem)

    limit = jnp.where(wid < NFULL, EPW, jnp.where(wid == NFULL, ETAIL, 0))
    zeros16 = jnp.zeros((16,), F32)

    def zero_body(i, carry):
        for j in range(NP // 16):
            p_loc[i, pl.ds(j * 16, 16)] = zeros16
            q_loc[i, pl.ds(j * 16, 16)] = zeros16
        return carry

    lax.fori_loop(0, NP, zero_body, 0, unroll=4)

    @pl.when(wid < NFULL)
    def _():
        pltpu.make_async_copy(srcrow.at[pl.ds(base, EPW)], src_v, sem).wait()
        pltpu.make_async_copy(dstrow.at[pl.ds(base, EPW)], dst_v, sem).wait()
        pltpu.make_async_copy(earow.at[pl.ds(base, EPW)], ea_v, sem).wait()

    @pl.when(wid == NFULL)
    def _():
        pltpu.make_async_copy(srcrow.at[pl.ds(base, ETDMA)],
                              src_v.at[pl.ds(0, ETDMA)], sem).wait()
        pltpu.make_async_copy(dstrow.at[pl.ds(base, ETDMA)],
                              dst_v.at[pl.ds(0, ETDMA)], sem).wait()
        pltpu.make_async_copy(earow.at[pl.ds(base, ETDMA)],
                              ea_v.at[pl.ds(0, ETDMA)], sem).wait()

    pltpu.make_async_copy(a2_h.at[0], ai_v, sem).wait()
    pltpu.make_async_copy(a2_h.at[1], aj_v, sem).wait()
    pltpu.make_async_copy(c_h.at[0], c_v, sem).wait()
    pltpu.make_async_copy(c_h.at[1], b_v, sem).wait()

    cv = c_v[...]
    bv = b_v[...]
    lane = lax.iota(jnp.int32, 16)

    def p2_body(g, carry):
        valid = (lane + g * 16) < limit
        d = jnp.where(valid, dst_v[pl.ds(g * 16, 16)], 0)
        s = jnp.where(valid, src_v[pl.ds(g * 16, 16)], 0)
        e = ea_v[pl.ds(g * 16, 16)]
        ai = plsc.load_gather(ai_v, [d])
        aj = plsc.load_gather(aj_v, [s])
        al = ai + aj + e * cv
        al = jnp.where(al >= 0.0, al, 0.2 * al)
        p0 = jnp.exp(al - bv)
        p = jnp.where(valid, p0, 0.0)
        q = jnp.where(valid, p0 * e, 0.0)
        k = jnp.where(valid, d * NP + s, NP * NP - 1)
        ks, perm = plsc.sort_key_val(k, lane)
        ps = _take(p, perm)
        qs = _take(q, perm)
        prev = _take(ks, jnp.maximum(lane - 1, 0))
        nxt = _take(ks, jnp.minimum(lane + 1, 15))
        is_start = (lane == 0) | (ks != prev)
        is_end = (lane == 15) | (ks != nxt)
        tp = plsc.cumsum(ps)
        tq = plsc.cumsum(qs)
        startlane = plsc.cummax(jnp.where(is_start, lane, 0))
        runp = tp - (_take(tp, startlane) - _take(ps, startlane))
        runq = tq - (_take(tq, startlane) - _take(qs, startlane))
        kd = ks // NP
        kc = ks - kd * NP
        plsc.addupdate_scatter(p_loc, [kd, kc], runp, mask=is_end)
        plsc.addupdate_scatter(q_loc, [kd, kc], runq, mask=is_end)
        return carry

    lax.fori_loop(0, G, p2_body, 0, unroll=4)

    pltpu.sync_copy(p_loc, pout.at[wid])
    pltpu.sync_copy(q_loc, qout.at[wid])


@functools.partial(
    pl.kernel,
    out_type=[
        jax.ShapeDtypeStruct((NW, NP, NP), F32),
        jax.ShapeDtypeStruct((NW, NP, NP), F32),
    ],
    mesh=plsc.VectorSubcoreMesh(core_axis_name="c", subcore_axis_name="s"),
    compiler_params=pltpu.CompilerParams(needs_layout_passes=False),
    scratch_types=[
        pltpu.VMEM((EPW,), jnp.int32),
        pltpu.VMEM((EPW,), jnp.int32),
        pltpu.VMEM((EPW,), F32),
        pltpu.VMEM((NP,), F32),
        pltpu.VMEM((NP,), F32),
        pltpu.VMEM((16,), F32),
        pltpu.VMEM((16,), F32),
        pltpu.VMEM((NP, NP), F32),
        pltpu.VMEM((NP, NP), F32),
        pltpu.SemaphoreType.DMA,
    ],
)
def _sc_edge(ei_h, ea_h, a2_h, c_h, pout, qout,
             src_v, dst_v, ea_v, ai_v, aj_v, c_v, b_v, p_loc, q_loc,
             sem):
    _sc_body(ei_h, ea_h, a2_h, c_h, pout, qout,
             src_v, dst_v, ea_v, ai_v, aj_v, c_v, b_v, p_loc, q_loc,
             sem)


def _tc2_body(p_ref, q_ref, xt_ref, eu_ref, b_ref, out_ref):
    P = jnp.sum(p_ref[...], axis=0)
    Q = jnp.sum(q_ref[...], axis=0)
    denom = jnp.sum(P, axis=1, keepdims=True) + 1e-16
    qn = jnp.sum(Q, axis=1, keepdims=True) / denom
    A = jnp.dot(P, xt_ref[...], preferred_element_type=F32) / denom
    eu = eu_ref[...]
    out = jnp.dot(A, eu[:C, :], preferred_element_type=F32)
    res = out + qn * eu[C:C + 1, :] + b_ref[...][None, :]
    out_ref[0] = res[:N, :]


def _tc2(pparts, qparts, xt, eu, b):
    return pl.pallas_call(
        _tc2_body,
        out_shape=jax.ShapeDtypeStruct((1, N, C), F32),
    )(pparts, qparts, xt, eu, b)


def kernel(x, edge_index, edge_attr, W, att, edge_update1, bias):
    ea_row = edge_attr.mT
    xt, a2, c2 = _tc1(x, W, att, ea_row)

    pparts, qparts = _sc_edge(edge_index, ea_row, a2, c2)

    return _tc2(pparts, qparts, xt, edge_update1, bias)
